# baseline (speedup 1.0000x reference)
import functools

import jax
import jax.numpy as jnp
from jax import lax
from jax.experimental import pallas as pl
from jax.experimental.pallas import tpu as pltpu
from jax.experimental.pallas import tpu_sc as plsc

_VOCAB = 100000
_D = 128
_SEQ = 128
_L = 122
_START = 99998
_END = 99999
_B = 1024
_CLIP = 512

_NC = 2
_NS = 16
_NW = _NC * _NS
_BPW = _B // _NW
_NBUF = 6
_LOOK = _NBUF - 1


def _proj_body(ex_ref, tg_ref, w_ref, b_ref, pos_ref, ids_ref, o_ref, idx_ref):
    w = w_ref[...]
    o_ref[:, 0, :] = jax.lax.dot_general(
        ex_ref[...], w, (((1,), (1,)), ((), ())),
        preferred_element_type=jnp.float32) + (b_ref[...] + pos_ref[pl.ds(3, 1), :])
    o_ref[:, 1, :] = jax.lax.dot_general(
        tg_ref[...], w, (((1,), (1,)), ((), ())),
        preferred_element_type=jnp.float32) + (b_ref[...] + pos_ref[pl.ds(6, 1), :])
    tb = ids_ref.shape[0]
    idx_ref[...] = jnp.concatenate(
        [ids_ref[...], jnp.zeros((tb, _SEQ - _L), jnp.int32)], axis=1)


def _proj(ex, tg, w, b, pos, ids):
    grid = 4
    tb = _B // grid
    return pl.pallas_call(
        _proj_body,
        grid=(grid,),
        in_specs=[
            pl.BlockSpec((tb, _CLIP), lambda i: (i, 0)),
            pl.BlockSpec((tb, _CLIP), lambda i: (i, 0)),
            pl.BlockSpec((_D, _CLIP), lambda i: (0, 0)),
            pl.BlockSpec((1, _D), lambda i: (0, 0)),
            pl.BlockSpec((8, _D), lambda i: (_SEQ // 8 - 1, 0)),
            pl.BlockSpec((tb, _L), lambda i: (i, 0)),
        ],
        out_specs=[
            pl.BlockSpec((tb, 2, _D), lambda i: (i, 0, 0)),
            pl.BlockSpec((tb, _SEQ), lambda i: (i, 0)),
        ],
        out_shape=[
            jax.ShapeDtypeStruct((_B, 2, _D), jnp.float32),
            jax.ShapeDtypeStruct((_B, _SEQ), jnp.int32),
        ],
    )(ex, tg, w, b, pos, ids)


_mesh = plsc.VectorSubcoreMesh(core_axis_name="c", subcore_axis_name="s")


@functools.partial(
    pl.kernel,
    mesh=_mesh,
    out_type=jax.ShapeDtypeStruct((_B, _SEQ, _D), jnp.float32),
    scratch_types=(
        [pltpu.VMEM((_BPW, _SEQ), jnp.int32),
         pltpu.VMEM((_BPW, 2, _D), jnp.float32),
         pltpu.VMEM((_SEQ, _D), jnp.float32),
         pltpu.VMEM((2, _D), jnp.float32)]
        + [pltpu.VMEM((_SEQ, _D), jnp.float32)] * _NBUF
        + [pltpu.SemaphoreType.DMA] * (2 * _NBUF)
    ),
)
def _sc_assemble(idx_hbm, table_hbm, proj_hbm, pos_hbm, out_hbm,
                 idx_v, proj_v, pos_v, se_v, buf0, buf1, buf2, buf3, buf4,
                 buf5, g0, g1, g2, g3, g4, g5, w0, w1, w2, w3, w4, w5):
    bufs = (buf0, buf1, buf2, buf3, buf4, buf5)
    gsems = (g0, g1, g2, g3, g4, g5)
    wsems = (w0, w1, w2, w3, w4, w5)
    wid = lax.axis_index("s") * _NC + lax.axis_index("c")
    base = wid * _BPW

    pltpu.sync_copy(idx_hbm.at[pl.ds(base, _BPW)], idx_v)
    pltpu.sync_copy(proj_hbm.at[pl.ds(base, _BPW)], proj_v)
    pltpu.sync_copy(pos_hbm, pos_v)
    pltpu.sync_copy(table_hbm.at[pl.ds(_START, 2)], se_v)

    for u in range(_NBUF):
        for c in range(_D // 16):
            sl = pl.ds(c * 16, 16)
            bufs[u][122, sl] = se_v[0, sl] + pos_v[122, sl]
            bufs[u][124, sl] = se_v[1, sl] + pos_v[124, sl]
            bufs[u][125, sl] = se_v[0, sl] + pos_v[125, sl]
            bufs[u][127, sl] = se_v[1, sl] + pos_v[127, sl]

    def fire(i, k):
        pltpu.async_copy(table_hbm.at[idx_v.at[i, pl.ds(0, _L)]],
                         bufs[k].at[pl.ds(0, _L)], gsems[k])

    def drain_gather(i, k):
        pltpu.make_async_copy(table_hbm.at[idx_v.at[i, pl.ds(0, _L)]],
                              bufs[k].at[pl.ds(0, _L)], gsems[k]).wait()

    for k in range(_LOOK):
        fire(k, k)

    def assemble(buf, i):
        def radd(r2, c2):
            for dr in range(2):
                r = r2 * 2 + dr
                for c in range(_D // 16):
                    sl = pl.ds(c * 16, 16)
                    buf[r, sl] = buf[r, sl] + pos_v[r, sl]
            return c2

        lax.fori_loop(0, _L // 2, radd, 0)
        for c in range(_D // 16):
            sl = pl.ds(c * 16, 16)
            buf[123, sl] = proj_v[i, 0, sl]
            buf[126, sl] = proj_v[i, 1, sl]

    def step(i, u):
        drain_gather(i, u)
        assemble(bufs[u], i)
        pltpu.async_copy(bufs[u], out_hbm.at[base + i], wsems[u])

        ku = (u + _LOOK) % _NBUF
        inext = i + _LOOK

        @pl.when(jnp.logical_and(i >= 1, inext < _BPW))
        def _drain():
            pltpu.make_async_copy(
                bufs[ku], out_hbm.at[base + i - 1], wsems[ku]).wait()

        @pl.when(inext < _BPW)
        def _fire():
            fire(inext, ku)

    def super_body(s, carry):
        for u in range(_NBUF):
            step(s * _NBUF + u, u)
        return carry

    lax.fori_loop(0, _BPW // _NBUF, super_body, 0)
    for t in range(_BPW % _NBUF):
        step((_BPW // _NBUF) * _NBUF + t, t)

    for u in range(_NBUF):
        pltpu.make_async_copy(bufs[u], out_hbm.at[base], wsems[u]).wait()


def kernel(full_prompt_ids, example_image_clip, target_image_clip,
           token_embed, clip_W, clip_b, pos_embed):
    ids = full_prompt_ids.astype(jnp.int32)
    proj, idx_full = _proj(example_image_clip, target_image_clip, clip_W,
                           clip_b.reshape(1, _D), pos_embed, ids)
    return _sc_assemble(idx_full, token_embed, proj, pos_embed)


# speedup vs baseline: 1.0211x; 1.0211x over previous
import functools

import jax
import jax.numpy as jnp
from jax import lax
from jax.experimental import pallas as pl
from jax.experimental.pallas import tpu as pltpu
from jax.experimental.pallas import tpu_sc as plsc

_VOCAB = 100000
_D = 128
_SEQ = 128
_L = 122
_START = 99998
_END = 99999
_B = 1024
_CLIP = 512

_NC = 2
_NS = 16
_NW = _NC * _NS
_BPW = _B // _NW
_NBUF = 4
_LOOK = _NBUF - 1


def _proj_body(ex_ref, tg_ref, w_ref, b_ref, pos_ref, ids_ref, o_ref, idx_ref):
    w = w_ref[...]
    o_ref[:, 0, :] = jax.lax.dot_general(
        ex_ref[...], w, (((1,), (1,)), ((), ())),
        preferred_element_type=jnp.float32) + (b_ref[...] + pos_ref[pl.ds(3, 1), :])
    o_ref[:, 1, :] = jax.lax.dot_general(
        tg_ref[...], w, (((1,), (1,)), ((), ())),
        preferred_element_type=jnp.float32) + (b_ref[...] + pos_ref[pl.ds(6, 1), :])
    tb = ids_ref.shape[0]
    idx_ref[...] = jnp.concatenate(
        [ids_ref[...], jnp.zeros((tb, _SEQ - _L), jnp.int32)], axis=1)


def _proj(ex, tg, w, b, pos, ids):
    grid = 4
    tb = _B // grid
    return pl.pallas_call(
        _proj_body,
        grid=(grid,),
        in_specs=[
            pl.BlockSpec((tb, _CLIP), lambda i: (i, 0)),
            pl.BlockSpec((tb, _CLIP), lambda i: (i, 0)),
            pl.BlockSpec((_D, _CLIP), lambda i: (0, 0)),
            pl.BlockSpec((1, _D), lambda i: (0, 0)),
            pl.BlockSpec((8, _D), lambda i: (_SEQ // 8 - 1, 0)),
            pl.BlockSpec((tb, _L), lambda i: (i, 0)),
        ],
        out_specs=[
            pl.BlockSpec((tb, 2, _D), lambda i: (i, 0, 0)),
            pl.BlockSpec((tb, _SEQ), lambda i: (i, 0)),
        ],
        out_shape=[
            jax.ShapeDtypeStruct((_B, 2, _D), jnp.float32),
            jax.ShapeDtypeStruct((_B, _SEQ), jnp.int32),
        ],
    )(ex, tg, w, b, pos, ids)


_mesh = plsc.VectorSubcoreMesh(core_axis_name="c", subcore_axis_name="s")


@functools.partial(
    pl.kernel,
    mesh=_mesh,
    out_type=jax.ShapeDtypeStruct((_B, _SEQ, _D), jnp.float32),
    scratch_types=(
        [pltpu.VMEM((_BPW, _SEQ), jnp.int32),
         pltpu.VMEM((_BPW, 2, _D), jnp.float32),
         pltpu.VMEM((_SEQ, _D), jnp.float32),
         pltpu.VMEM((2, _D), jnp.float32)]
        + [pltpu.VMEM((_SEQ, _D), jnp.float32)] * _NBUF
        + [pltpu.SemaphoreType.DMA] * (2 * _NBUF)
    ),
)
def _sc_assemble(idx_hbm, table_hbm, proj_hbm, pos_hbm, out_hbm,
                 idx_v, proj_v, pos_v, se_v, buf0, buf1, buf2, buf3,
                 g0, g1, g2, g3, w0, w1, w2, w3):
    bufs = (buf0, buf1, buf2, buf3)
    gsems = (g0, g1, g2, g3)
    wsems = (w0, w1, w2, w3)
    wid = lax.axis_index("s") * _NC + lax.axis_index("c")
    base = wid * _BPW

    pltpu.sync_copy(idx_hbm.at[pl.ds(base, _BPW)], idx_v)
    pltpu.sync_copy(proj_hbm.at[pl.ds(base, _BPW)], proj_v)
    pltpu.sync_copy(pos_hbm, pos_v)
    pltpu.sync_copy(table_hbm.at[pl.ds(_START, 2)], se_v)

    for u in range(_NBUF):
        for c in range(_D // 16):
            sl = pl.ds(c * 16, 16)
            bufs[u][122, sl] = se_v[0, sl] + pos_v[122, sl]
            bufs[u][124, sl] = se_v[1, sl] + pos_v[124, sl]
            bufs[u][125, sl] = se_v[0, sl] + pos_v[125, sl]
            bufs[u][127, sl] = se_v[1, sl] + pos_v[127, sl]

    def fire(i, k):
        pltpu.async_copy(table_hbm.at[idx_v.at[i, pl.ds(0, _L)]],
                         bufs[k].at[pl.ds(0, _L)], gsems[k])

    def drain_gather(i, k):
        pltpu.make_async_copy(table_hbm.at[idx_v.at[i, pl.ds(0, _L)]],
                              bufs[k].at[pl.ds(0, _L)], gsems[k]).wait()

    for k in range(_LOOK):
        fire(k, k)

    def assemble(buf, i):
        def radd(r2, c2):
            for dr in range(2):
                r = r2 * 2 + dr
                for c in range(_D // 16):
                    sl = pl.ds(c * 16, 16)
                    buf[r, sl] = buf[r, sl] + pos_v[r, sl]
            return c2

        lax.fori_loop(0, _L // 2, radd, 0)
        for c in range(_D // 16):
            sl = pl.ds(c * 16, 16)
            buf[123, sl] = proj_v[i, 0, sl]
            buf[126, sl] = proj_v[i, 1, sl]

    def step(i, u):
        drain_gather(i, u)
        assemble(bufs[u], i)
        pltpu.async_copy(bufs[u], out_hbm.at[base + i], wsems[u])

        ku = (u + _LOOK) % _NBUF
        inext = i + _LOOK

        @pl.when(jnp.logical_and(i >= 1, inext < _BPW))
        def _drain():
            pltpu.make_async_copy(
                bufs[ku], out_hbm.at[base + i - 1], wsems[ku]).wait()

        @pl.when(inext < _BPW)
        def _fire():
            fire(inext, ku)

    def super_body(s, carry):
        for u in range(_NBUF):
            step(s * _NBUF + u, u)
        return carry

    lax.fori_loop(0, _BPW // _NBUF, super_body, 0)
    for t in range(_BPW % _NBUF):
        step((_BPW // _NBUF) * _NBUF + t, t)

    for u in range(_NBUF):
        pltpu.make_async_copy(bufs[u], out_hbm.at[base], wsems[u]).wait()


def kernel(full_prompt_ids, example_image_clip, target_image_clip,
           token_embed, clip_W, clip_b, pos_embed):
    ids = full_prompt_ids.astype(jnp.int32)
    proj, idx_full = _proj(example_image_clip, target_image_clip, clip_W,
                           clip_b.reshape(1, _D), pos_embed, ids)
    return _sc_assemble(idx_full, token_embed, proj, pos_embed)


# speedup vs baseline: 1.0433x; 1.0218x over previous
import functools

import jax
import jax.numpy as jnp
from jax import lax
from jax.experimental import pallas as pl
from jax.experimental.pallas import tpu as pltpu
from jax.experimental.pallas import tpu_sc as plsc

_VOCAB = 100000
_D = 128
_SEQ = 128
_L = 122
_START = 99998
_END = 99999
_B = 1024
_CLIP = 512

_NC = 2
_NS = 16
_NW = _NC * _NS
_BPW = _B // _NW
_NBUF = 4
_LOOK = _NBUF - 1


def _proj_body(ex_ref, tg_ref, w_ref, b_ref, pos_ref, ids_ref, o_ref, idx_ref):
    w = w_ref[...]
    o_ref[:, 0, :] = jax.lax.dot_general(
        ex_ref[...], w, (((1,), (1,)), ((), ())),
        preferred_element_type=jnp.float32) + (b_ref[...] + pos_ref[pl.ds(3, 1), :])
    o_ref[:, 1, :] = jax.lax.dot_general(
        tg_ref[...], w, (((1,), (1,)), ((), ())),
        preferred_element_type=jnp.float32) + (b_ref[...] + pos_ref[pl.ds(6, 1), :])
    tb = ids_ref.shape[0]
    idx_ref[...] = jnp.concatenate(
        [ids_ref[...], jnp.zeros((tb, _SEQ - _L), jnp.int32)], axis=1)


def _proj(ex, tg, w, b, pos, ids):
    grid = 4
    tb = _B // grid
    return pl.pallas_call(
        _proj_body,
        grid=(grid,),
        in_specs=[
            pl.BlockSpec((tb, _CLIP), lambda i: (i, 0)),
            pl.BlockSpec((tb, _CLIP), lambda i: (i, 0)),
            pl.BlockSpec((_D, _CLIP), lambda i: (0, 0)),
            pl.BlockSpec((1, _D), lambda i: (0, 0)),
            pl.BlockSpec((8, _D), lambda i: (_SEQ // 8 - 1, 0)),
            pl.BlockSpec((tb, _L), lambda i: (i, 0)),
        ],
        out_specs=[
            pl.BlockSpec((tb, 2, _D), lambda i: (i, 0, 0)),
            pl.BlockSpec((tb, _SEQ), lambda i: (i, 0)),
        ],
        out_shape=[
            jax.ShapeDtypeStruct((_B, 2, _D), jnp.float32),
            jax.ShapeDtypeStruct((_B, _SEQ), jnp.int32),
        ],
    )(ex, tg, w, b, pos, ids)


_mesh = plsc.VectorSubcoreMesh(core_axis_name="c", subcore_axis_name="s")


@functools.partial(
    pl.kernel,
    mesh=_mesh,
    out_type=jax.ShapeDtypeStruct((_B, _SEQ, _D), jnp.float32),
    scratch_types=(
        [pltpu.VMEM((_BPW, _SEQ), jnp.int32),
         pltpu.VMEM((_BPW, 2, _D), jnp.float32),
         pltpu.VMEM((_SEQ, _D), jnp.float32),
         pltpu.VMEM((2, _D), jnp.float32)]
        + [pltpu.VMEM((_SEQ, _D), jnp.float32)] * _NBUF
        + [pltpu.SemaphoreType.DMA] * (2 * _NBUF)
    ),
)
def _sc_assemble(idx_hbm, table_hbm, proj_hbm, pos_hbm, out_hbm,
                 idx_v, proj_v, pos_v, se_v, buf0, buf1, buf2, buf3,
                 g0, g1, g2, g3, w0, w1, w2, w3):
    bufs = (buf0, buf1, buf2, buf3)
    gsems = (g0, g1, g2, g3)
    wsems = (w0, w1, w2, w3)
    wid = lax.axis_index("s") * _NC + lax.axis_index("c")
    base = wid * _BPW

    def fire(i, k):
        pltpu.async_copy(table_hbm.at[idx_v.at[i, pl.ds(0, _L)]],
                         bufs[k].at[pl.ds(0, _L)], gsems[k])

    def drain_gather(i, k):
        pltpu.make_async_copy(table_hbm.at[idx_v.at[i, pl.ds(0, _L)]],
                              bufs[k].at[pl.ds(0, _L)], gsems[k]).wait()

    pltpu.sync_copy(idx_hbm.at[pl.ds(base, _BPW)], idx_v)
    for k in range(_LOOK):
        fire(k, k)

    pltpu.sync_copy(proj_hbm.at[pl.ds(base, _BPW)], proj_v)
    pltpu.sync_copy(pos_hbm, pos_v)
    pltpu.sync_copy(table_hbm.at[pl.ds(_START, 2)], se_v)

    for u in range(_NBUF):
        for c in range(_D // 16):
            sl = pl.ds(c * 16, 16)
            bufs[u][122, sl] = se_v[0, sl] + pos_v[122, sl]
            bufs[u][124, sl] = se_v[1, sl] + pos_v[124, sl]
            bufs[u][125, sl] = se_v[0, sl] + pos_v[125, sl]
            bufs[u][127, sl] = se_v[1, sl] + pos_v[127, sl]

    def assemble(buf, i):
        def radd(r2, c2):
            for dr in range(2):
                r = r2 * 2 + dr
                for c in range(_D // 16):
                    sl = pl.ds(c * 16, 16)
                    buf[r, sl] = buf[r, sl] + pos_v[r, sl]
            return c2

        lax.fori_loop(0, _L // 2, radd, 0)
        for c in range(_D // 16):
            sl = pl.ds(c * 16, 16)
            buf[123, sl] = proj_v[i, 0, sl]
            buf[126, sl] = proj_v[i, 1, sl]

    def step(i, u):
        drain_gather(i, u)
        assemble(bufs[u], i)
        pltpu.async_copy(bufs[u], out_hbm.at[base + i], wsems[u])

        ku = (u + _LOOK) % _NBUF
        inext = i + _LOOK

        @pl.when(jnp.logical_and(i >= 1, inext < _BPW))
        def _drain():
            pltpu.make_async_copy(
                bufs[ku], out_hbm.at[base + i - 1], wsems[ku]).wait()

        @pl.when(inext < _BPW)
        def _fire():
            fire(inext, ku)

    def super_body(s, carry):
        for u in range(_NBUF):
            step(s * _NBUF + u, u)
        return carry

    lax.fori_loop(0, _BPW // _NBUF, super_body, 0)
    for t in range(_BPW % _NBUF):
        step((_BPW // _NBUF) * _NBUF + t, t)

    for u in range(_NBUF):
        pltpu.make_async_copy(bufs[u], out_hbm.at[base], wsems[u]).wait()


def kernel(full_prompt_ids, example_image_clip, target_image_clip,
           token_embed, clip_W, clip_b, pos_embed):
    ids = full_prompt_ids.astype(jnp.int32)
    proj, idx_full = _proj(example_image_clip, target_image_clip, clip_W,
                           clip_b.reshape(1, _D), pos_embed, ids)
    return _sc_assemble(idx_full, token_embed, proj, pos_embed)


# speedup vs baseline: 1.0494x; 1.0059x over previous
import functools

import jax
import jax.numpy as jnp
from jax import lax
from jax.experimental import pallas as pl
from jax.experimental.pallas import tpu as pltpu
from jax.experimental.pallas import tpu_sc as plsc

_VOCAB = 100000
_D = 128
_SEQ = 128
_L = 122
_START = 99998
_END = 99999
_B = 1024
_CLIP = 512

_NC = 2
_NS = 16
_NW = _NC * _NS
_BPW = _B // _NW
_NBUF = 4
_LOOK = _NBUF - 1


def _proj_body(ex_ref, tg_ref, w_ref, b_ref, pos_ref, ids_ref, o_ref, idx_ref):
    w = w_ref[...]
    o_ref[:, 0, :] = jax.lax.dot_general(
        ex_ref[...], w, (((1,), (1,)), ((), ())),
        preferred_element_type=jnp.float32) + (b_ref[...] + pos_ref[pl.ds(3, 1), :])
    o_ref[:, 1, :] = jax.lax.dot_general(
        tg_ref[...], w, (((1,), (1,)), ((), ())),
        preferred_element_type=jnp.float32) + (b_ref[...] + pos_ref[pl.ds(6, 1), :])
    tb = ids_ref.shape[0]
    idx_ref[...] = jnp.concatenate(
        [ids_ref[...], jnp.zeros((tb, _SEQ - _L), jnp.int32)], axis=1)


def _proj(ex, tg, w, b, pos, ids):
    grid = 1
    tb = _B // grid
    return pl.pallas_call(
        _proj_body,
        grid=(grid,),
        in_specs=[
            pl.BlockSpec((tb, _CLIP), lambda i: (i, 0)),
            pl.BlockSpec((tb, _CLIP), lambda i: (i, 0)),
            pl.BlockSpec((_D, _CLIP), lambda i: (0, 0)),
            pl.BlockSpec((1, _D), lambda i: (0, 0)),
            pl.BlockSpec((8, _D), lambda i: (_SEQ // 8 - 1, 0)),
            pl.BlockSpec((tb, _L), lambda i: (i, 0)),
        ],
        out_specs=[
            pl.BlockSpec((tb, 2, _D), lambda i: (i, 0, 0)),
            pl.BlockSpec((tb, _SEQ), lambda i: (i, 0)),
        ],
        out_shape=[
            jax.ShapeDtypeStruct((_B, 2, _D), jnp.float32),
            jax.ShapeDtypeStruct((_B, _SEQ), jnp.int32),
        ],
    )(ex, tg, w, b, pos, ids)


_mesh = plsc.VectorSubcoreMesh(core_axis_name="c", subcore_axis_name="s")


@functools.partial(
    pl.kernel,
    mesh=_mesh,
    out_type=jax.ShapeDtypeStruct((_B, _SEQ, _D), jnp.float32),
    scratch_types=(
        [pltpu.VMEM((_BPW, _SEQ), jnp.int32),
         pltpu.VMEM((_BPW, 2, _D), jnp.float32),
         pltpu.VMEM((_SEQ, _D), jnp.float32),
         pltpu.VMEM((2, _D), jnp.float32)]
        + [pltpu.VMEM((_SEQ, _D), jnp.float32)] * _NBUF
        + [pltpu.SemaphoreType.DMA] * (2 * _NBUF)
    ),
)
def _sc_assemble(idx_hbm, table_hbm, proj_hbm, pos_hbm, out_hbm,
                 idx_v, proj_v, pos_v, se_v, buf0, buf1, buf2, buf3,
                 g0, g1, g2, g3, w0, w1, w2, w3):
    bufs = (buf0, buf1, buf2, buf3)
    gsems = (g0, g1, g2, g3)
    wsems = (w0, w1, w2, w3)
    wid = lax.axis_index("s") * _NC + lax.axis_index("c")
    base = wid * _BPW

    def fire(i, k):
        pltpu.async_copy(table_hbm.at[idx_v.at[i, pl.ds(0, _L)]],
                         bufs[k].at[pl.ds(0, _L)], gsems[k])

    def drain_gather(i, k):
        pltpu.make_async_copy(table_hbm.at[idx_v.at[i, pl.ds(0, _L)]],
                              bufs[k].at[pl.ds(0, _L)], gsems[k]).wait()

    pltpu.sync_copy(idx_hbm.at[pl.ds(base, _BPW)], idx_v)
    for k in range(_LOOK):
        fire(k, k)

    pltpu.sync_copy(proj_hbm.at[pl.ds(base, _BPW)], proj_v)
    pltpu.sync_copy(pos_hbm, pos_v)
    pltpu.sync_copy(table_hbm.at[pl.ds(_START, 2)], se_v)

    for u in range(_NBUF):
        for c in range(_D // 16):
            sl = pl.ds(c * 16, 16)
            bufs[u][122, sl] = se_v[0, sl] + pos_v[122, sl]
            bufs[u][124, sl] = se_v[1, sl] + pos_v[124, sl]
            bufs[u][125, sl] = se_v[0, sl] + pos_v[125, sl]
            bufs[u][127, sl] = se_v[1, sl] + pos_v[127, sl]

    def assemble(buf, i):
        def radd(r2, c2):
            for dr in range(2):
                r = r2 * 2 + dr
                for c in range(_D // 16):
                    sl = pl.ds(c * 16, 16)
                    buf[r, sl] = buf[r, sl] + pos_v[r, sl]
            return c2

        lax.fori_loop(0, _L // 2, radd, 0)
        for c in range(_D // 16):
            sl = pl.ds(c * 16, 16)
            buf[123, sl] = proj_v[i, 0, sl]
            buf[126, sl] = proj_v[i, 1, sl]

    def step(i, u):
        drain_gather(i, u)
        assemble(bufs[u], i)
        pltpu.async_copy(bufs[u], out_hbm.at[base + i], wsems[u])

        ku = (u + _LOOK) % _NBUF
        inext = i + _LOOK

        @pl.when(jnp.logical_and(i >= 1, inext < _BPW))
        def _drain():
            pltpu.make_async_copy(
                bufs[ku], out_hbm.at[base + i - 1], wsems[ku]).wait()

        @pl.when(inext < _BPW)
        def _fire():
            fire(inext, ku)

    def super_body(s, carry):
        for u in range(_NBUF):
            step(s * _NBUF + u, u)
        return carry

    lax.fori_loop(0, _BPW // _NBUF, super_body, 0)
    for t in range(_BPW % _NBUF):
        step((_BPW // _NBUF) * _NBUF + t, t)

    for u in range(_NBUF):
        pltpu.make_async_copy(bufs[u], out_hbm.at[base], wsems[u]).wait()


def kernel(full_prompt_ids, example_image_clip, target_image_clip,
           token_embed, clip_W, clip_b, pos_embed):
    ids = full_prompt_ids.astype(jnp.int32)
    proj, idx_full = _proj(example_image_clip, target_image_clip, clip_W,
                           clip_b.reshape(1, _D), pos_embed, ids)
    return _sc_assemble(idx_full, token_embed, proj, pos_embed)
